# baseline (speedup 1.0000x reference)
import functools

import jax
import jax.numpy as jnp
from jax import lax
from jax.experimental import pallas as pl
from jax.experimental.pallas import tpu as pltpu
from jax.experimental.pallas import tpu_sc as plsc

_N = 36864
_D = 64
_K = 1024
_CC = 0.25
_BLK = 512
_GRID = _N // _BLK

_NC = 2
_NS = 16
_NW = _NC * _NS
_BPW = _N // _NW


def _tc_body(x_ref, w_ref, idx_ref, loss_ref, perp_ref, wsq_ref, counts_ref,
             acc_ref):
    i = pl.program_id(0)
    w = w_ref[...]

    @pl.when(i == 0)
    def _init():
        ones = jnp.ones((1, _D), jnp.float32)
        wsq_ref[...] = lax.dot_general(
            ones, w * w, (((1,), (1,)), ((), ())),
            preferred_element_type=jnp.float32,
            precision=lax.Precision.HIGHEST)
        counts_ref[...] = jnp.zeros_like(counts_ref)
        acc_ref[...] = jnp.zeros_like(acc_ref)

    x = x_ref[...]
    xsq = jnp.sum(x * x, axis=1, keepdims=True)
    dots = lax.dot_general(
        x, w, (((1,), (1,)), ((), ())),
        preferred_element_type=jnp.float32)
    d = xsq + wsq_ref[...] - 2.0 * dots
    mind = jnp.min(d, axis=1, keepdims=True)
    iota = lax.broadcasted_iota(jnp.int32, (_BLK, _K), 1)
    idx = jnp.min(jnp.where(d == mind, iota, _K), axis=1, keepdims=True)
    idx_ref[...] = idx

    oh = (iota == idx).astype(jnp.float32)
    counts_ref[...] += jnp.sum(oh.reshape(_BLK // 8, 8, _K), axis=0)
    acc_ref[...] += jnp.sum(mind, axis=0, keepdims=True)

    @pl.when(i == _GRID - 1)
    def _fin():
        mse = acc_ref[...] * (1.0 / (_N * _D))
        loss_ref[...] = mse + _CC * mse
        p = jnp.sum(counts_ref[...], axis=0, keepdims=True) * (1.0 / _N)
        ent = jnp.sum(p * jnp.log(p + 1e-10), axis=1, keepdims=True)
        perp_ref[...] = jnp.exp(-ent)


_tc_quantize = pl.pallas_call(
    _tc_body,
    grid=(_GRID,),
    in_specs=[
        pl.BlockSpec((_BLK, _D), lambda i: (i, 0)),
        pl.BlockSpec((_K, _D), lambda i: (0, 0)),
    ],
    out_specs=[
        pl.BlockSpec((_BLK, 1), lambda i: (i, 0)),
        pl.BlockSpec((1, 1), lambda i: (0, 0)),
        pl.BlockSpec((1, 1), lambda i: (0, 0)),
    ],
    out_shape=[
        jax.ShapeDtypeStruct((_N, 1), jnp.int32),
        jax.ShapeDtypeStruct((1, 1), jnp.float32),
        jax.ShapeDtypeStruct((1, 1), jnp.float32),
    ],
    scratch_shapes=[
        pltpu.VMEM((1, _K), jnp.float32),
        pltpu.VMEM((8, _K), jnp.float32),
        pltpu.VMEM((1, 1), jnp.float32),
    ],
)


@functools.cache
def _make_sc_gather():
    mesh = plsc.VectorSubcoreMesh(core_axis_name="c", subcore_axis_name="s")

    @functools.partial(
        pl.kernel,
        mesh=mesh,
        out_type=jax.ShapeDtypeStruct((_N, _D), jnp.float32),
        scratch_types=[
            pltpu.VMEM((_BPW,), jnp.int32),
            pltpu.VMEM((_BPW, _D), jnp.float32),
            pltpu.SemaphoreType.DMA,
        ],
        compiler_params=pltpu.CompilerParams(use_tc_tiling_on_sc=False),
    )
    def _sc_gather(w_hbm, idx_hbm, out_hbm, idx_v, rows_v, sem):
        wid = lax.axis_index("s") * _NC + lax.axis_index("c")
        base = wid * _BPW
        pltpu.sync_copy(idx_hbm.at[pl.ds(base, _BPW)], idx_v)
        pltpu.async_copy(w_hbm.at[idx_v], rows_v, sem).wait()
        pltpu.sync_copy(rows_v, out_hbm.at[pl.ds(base, _BPW)])

    return _sc_gather


def kernel(inputs, W):
    idx2, loss11, perp11 = _tc_quantize(inputs, W)
    idx = idx2.reshape(_N)
    quantized = _make_sc_gather()(W, idx)
    return (quantized, loss11[0, 0], perp11[0, 0], idx)


# speedup vs baseline: 1.1045x; 1.1045x over previous
import functools

import jax
import jax.numpy as jnp
from jax import lax
from jax.experimental import pallas as pl
from jax.experimental.pallas import tpu as pltpu
from jax.experimental.pallas import tpu_sc as plsc

_N = 36864
_D = 64
_K = 1024
_CC = 0.25
_BLK = 1024
_GRID = _N // _BLK

_NC = 2
_NS = 16
_NW = _NC * _NS
_BPW = _N // _NW


def _tc_body(x_ref, w_ref, idx_ref, loss_ref, perp_ref, wsq_ref, counts_ref,
             acc_ref):
    i = pl.program_id(0)
    w = w_ref[...]

    @pl.when(i == 0)
    def _init():
        ones = jnp.ones((1, _D), jnp.float32)
        wsq_ref[...] = lax.dot_general(
            ones, w * w, (((1,), (1,)), ((), ())),
            preferred_element_type=jnp.float32,
            precision=lax.Precision.HIGHEST)
        counts_ref[...] = jnp.zeros_like(counts_ref)
        acc_ref[...] = jnp.zeros_like(acc_ref)

    x = x_ref[...]
    xsq = jnp.sum(x * x, axis=1, keepdims=True)
    dots2 = lax.dot_general(
        x + x, w, (((1,), (1,)), ((), ())),
        preferred_element_type=jnp.float32)
    d = (xsq + wsq_ref[...]) - dots2
    mind = jnp.min(d, axis=1, keepdims=True)
    iota = lax.broadcasted_iota(jnp.int32, (_BLK, _K), 1).astype(jnp.float32)
    idxf = jnp.min(jnp.where(d == mind, iota, float(_K)), axis=1,
                   keepdims=True)
    idx_ref[...] = idxf.astype(jnp.int32)

    oh = jnp.where(iota == idxf, 1.0, 0.0)
    counts_ref[...] += lax.dot_general(
        jnp.ones((1, _BLK), jnp.float32), oh, (((1,), (0,)), ((), ())),
        preferred_element_type=jnp.float32)
    acc_ref[...] += jnp.sum(mind, axis=0, keepdims=True)

    @pl.when(i == _GRID - 1)
    def _fin():
        mse = acc_ref[...] * (1.0 / (_N * _D))
        loss_ref[...] = mse + _CC * mse
        p = counts_ref[...] * (1.0 / _N)
        ent = jnp.sum(p * jnp.log(p + 1e-10), axis=1, keepdims=True)
        perp_ref[...] = jnp.exp(-ent)


_tc_quantize = pl.pallas_call(
    _tc_body,
    grid=(_GRID,),
    in_specs=[
        pl.BlockSpec((_BLK, _D), lambda i: (i, 0)),
        pl.BlockSpec((_K, _D), lambda i: (0, 0)),
    ],
    out_specs=[
        pl.BlockSpec((_BLK, 1), lambda i: (i, 0)),
        pl.BlockSpec((1, 1), lambda i: (0, 0)),
        pl.BlockSpec((1, 1), lambda i: (0, 0)),
    ],
    out_shape=[
        jax.ShapeDtypeStruct((_N, 1), jnp.int32),
        jax.ShapeDtypeStruct((1, 1), jnp.float32),
        jax.ShapeDtypeStruct((1, 1), jnp.float32),
    ],
    scratch_shapes=[
        pltpu.VMEM((1, _K), jnp.float32),
        pltpu.VMEM((1, _K), jnp.float32),
        pltpu.VMEM((1, 1), jnp.float32),
    ],
)


@functools.cache
def _make_sc_gather():
    mesh = plsc.VectorSubcoreMesh(core_axis_name="c", subcore_axis_name="s")

    @functools.partial(
        pl.kernel,
        mesh=mesh,
        out_type=jax.ShapeDtypeStruct((_N, _D), jnp.float32),
        scratch_types=[
            pltpu.VMEM((_BPW,), jnp.int32),
            pltpu.VMEM((_BPW, _D), jnp.float32),
            pltpu.SemaphoreType.DMA,
        ],
        compiler_params=pltpu.CompilerParams(use_tc_tiling_on_sc=False),
    )
    def _sc_gather(w_hbm, idx_hbm, out_hbm, idx_v, rows_v, sem):
        wid = lax.axis_index("s") * _NC + lax.axis_index("c")
        base = wid * _BPW
        pltpu.sync_copy(idx_hbm.at[pl.ds(base, _BPW)], idx_v)
        pltpu.async_copy(w_hbm.at[idx_v], rows_v, sem).wait()
        pltpu.sync_copy(rows_v, out_hbm.at[pl.ds(base, _BPW)])

    return _sc_gather


def kernel(inputs, W):
    idx2, loss11, perp11 = _tc_quantize(inputs, W)
    idx = idx2.reshape(_N)
    quantized = _make_sc_gather()(W, idx)
    return (quantized, loss11[0, 0], perp11[0, 0], idx)


# speedup vs baseline: 1.2710x; 1.1507x over previous
import functools

import jax
import jax.numpy as jnp
from jax import lax
from jax.experimental import pallas as pl
from jax.experimental.pallas import tpu as pltpu
from jax.experimental.pallas import tpu_sc as plsc

_N = 36864
_D = 64
_K = 1024
_CC = 0.25
_BLK = 1024
_GRID = _N // _BLK

_NC = 2
_NS = 16
_NW = _NC * _NS
_BPW = _N // _NW


def _tc_body(xt_ref, w_ref, idx_ref, loss_ref, perp_ref, wsq_ref, iota_ref,
             counts_ref, acc_ref):
    i = pl.program_id(0)
    w = w_ref[...]

    @pl.when(i == 0)
    def _init():
        wsq_ref[...] = jnp.sum(w * w, axis=1, keepdims=True)
        iota_ref[...] = lax.broadcasted_iota(
            jnp.int32, (_K, 1), 0).astype(jnp.float32)
        counts_ref[...] = jnp.zeros_like(counts_ref)
        acc_ref[...] = jnp.zeros_like(acc_ref)

    xt = xt_ref[...]
    xsq = jnp.sum(xt * xt, axis=0, keepdims=True)
    dots2 = lax.dot_general(
        w + w, xt, (((1,), (0,)), ((), ())),
        preferred_element_type=jnp.float32)
    d = (xsq + wsq_ref[...]) - dots2
    mind = jnp.min(d, axis=0, keepdims=True)
    iota = iota_ref[...] + jnp.zeros((_K, _BLK), jnp.float32)
    idxf = jnp.min(jnp.where(d == mind, iota, float(_K)), axis=0,
                   keepdims=True)
    idx_ref[...] = idxf.astype(jnp.int32).reshape(_BLK)

    oh = jnp.where(iota == idxf, 1.0, 0.0)
    counts_ref[...] += lax.dot_general(
        oh, jnp.ones((_BLK, 1), jnp.float32), (((1,), (0,)), ((), ())),
        preferred_element_type=jnp.float32)
    acc_ref[...] += jnp.sum(mind, axis=1, keepdims=True)

    @pl.when(i == _GRID - 1)
    def _fin():
        mse = acc_ref[...] * (1.0 / (_N * _D))
        loss_ref[...] = mse + _CC * mse
        p = counts_ref[...] * (1.0 / _N)
        ent = jnp.sum(p * jnp.log(p + 1e-10), axis=0, keepdims=True)
        perp_ref[...] = jnp.exp(-ent)


_tc_quantize = pl.pallas_call(
    _tc_body,
    grid=(_GRID,),
    in_specs=[
        pl.BlockSpec((_D, _BLK), lambda i: (0, i)),
        pl.BlockSpec((_K, _D), lambda i: (0, 0)),
    ],
    out_specs=[
        pl.BlockSpec((_BLK,), lambda i: (i,)),
        pl.BlockSpec((1, 1), lambda i: (0, 0)),
        pl.BlockSpec((1, 1), lambda i: (0, 0)),
    ],
    out_shape=[
        jax.ShapeDtypeStruct((_N,), jnp.int32),
        jax.ShapeDtypeStruct((1, 1), jnp.float32),
        jax.ShapeDtypeStruct((1, 1), jnp.float32),
    ],
    scratch_shapes=[
        pltpu.VMEM((_K, 1), jnp.float32),
        pltpu.VMEM((_K, 1), jnp.float32),
        pltpu.VMEM((_K, 1), jnp.float32),
        pltpu.VMEM((1, 1), jnp.float32),
    ],
)


@functools.cache
def _make_sc_gather():
    mesh = plsc.VectorSubcoreMesh(core_axis_name="c", subcore_axis_name="s")

    @functools.partial(
        pl.kernel,
        mesh=mesh,
        out_type=jax.ShapeDtypeStruct((_NW, _BPW, _D), jnp.float32),
        scratch_types=[
            pltpu.VMEM((_BPW,), jnp.int32),
            pltpu.VMEM((_BPW, _D), jnp.float32),
            pltpu.SemaphoreType.DMA,
        ],
        compiler_params=pltpu.CompilerParams(use_tc_tiling_on_sc=False),
    )
    def _sc_gather(w_hbm, idx_hbm, out_hbm, idx_v, rows_v, sem):
        wid = lax.axis_index("s") * _NC + lax.axis_index("c")
        base = wid * _BPW
        pltpu.sync_copy(idx_hbm.at[pl.ds(base, _BPW)], idx_v)
        pltpu.async_copy(w_hbm.at[idx_v], rows_v, sem).wait()
        pltpu.sync_copy(rows_v, out_hbm.at[wid])

    return _sc_gather


def kernel(inputs, W):
    idx, loss11, perp11 = _tc_quantize(inputs.T, W)
    q3 = _make_sc_gather()(W, idx)
    quantized = q3.reshape(_N, _D)
    return (quantized, loss11[0, 0], perp11[0, 0], idx)
